# baseline (speedup 1.0000x reference)
import functools

import jax
import jax.numpy as jnp
from jax import lax
from jax.experimental import pallas as pl
from jax.experimental.pallas import tpu as pltpu
from jax.experimental.pallas import tpu_sc as plsc

NW = 32
CHUNK = 128


def _make_gather_kernel(N, D):
    npw = N // NW
    nch = npw // CHUNK
    mesh = plsc.VectorSubcoreMesh(core_axis_name="c", subcore_axis_name="s")

    @functools.partial(
        pl.kernel,
        mesh=mesh,
        compiler_params=pltpu.CompilerParams(use_tc_tiling_on_sc=False),
        out_type=(
            jax.ShapeDtypeStruct((N, D), jnp.float32),
            jax.ShapeDtypeStruct((N,), jnp.float32),
        ),
        scratch_types=[
            pltpu.VMEM((CHUNK,), jnp.int32),
            pltpu.VMEM((CHUNK, D), jnp.float32),
            pltpu.VMEM((CHUNK,), jnp.float32),
            pltpu.SemaphoreType.DMA,
            pltpu.SemaphoreType.DMA,
        ],
    )
    def gather(emb2_hbm, emb1_hbm, idx_hbm, out2_hbm, out1_hbm,
               idx_v, rows_v, vals_v, sem2, sem1):
        wid = lax.axis_index("s") * 2 + lax.axis_index("c")
        base_w = wid * npw

        def body(c, carry):
            base = pl.multiple_of(base_w + c * CHUNK, CHUNK)
            pltpu.sync_copy(idx_hbm.at[pl.ds(base, CHUNK)], idx_v)
            cp2 = pltpu.async_copy(emb2_hbm.at[idx_v], rows_v, sem2)
            cp1 = pltpu.async_copy(emb1_hbm.at[idx_v], vals_v, sem1)
            cp2.wait()
            cp1.wait()
            pltpu.sync_copy(rows_v, out2_hbm.at[pl.ds(base, CHUNK)])
            pltpu.sync_copy(vals_v, out1_hbm.at[pl.ds(base, CHUNK)])
            return carry

        lax.fori_loop(0, nch, body, 0)

    return gather


def _dense_body(cont_ref, cat_ref, g1_ref, Wc_ref, W0c_ref, W0e_ref, b0_ref,
                W1_ref, b1_ref, Wh_ref, S_ref, sc_ref, out_ref):
    prec = lax.Precision.HIGHEST
    cont = cont_ref[...]
    cat = cat_ref[...]
    b_cont = sc_ref[0]
    b_out = sc_ref[1]
    w_fm = sc_ref[2]
    fm1 = (jnp.dot(cont, Wc_ref[...], precision=prec)
           + jnp.sum(g1_ref[...], axis=1, keepdims=True) + b_cont)
    sum_emb = jnp.dot(cat, S_ref[...], precision=prec)
    fm2 = 0.5 * (jnp.sum(sum_emb * sum_emb, axis=1, keepdims=True)
                 - jnp.sum(cat * cat, axis=1, keepdims=True))
    fm = fm1 + fm2
    h = jnp.maximum(jnp.dot(cont, W0c_ref[...], precision=prec)
                    + jnp.dot(cat, W0e_ref[...], precision=prec)
                    + b0_ref[...], 0.0)
    h = jnp.maximum(jnp.dot(h, W1_ref[...], precision=prec) + b1_ref[...], 0.0)
    out_ref[...] = (jnp.dot(h, Wh_ref[...], precision=prec)
                    + fm * w_fm + b_out)


def kernel(continuous, categorical, emb1, emb2, W_cont, b_cont, W0, b0, W1,
           b1, W_out, b_out):
    F, V, D = emb2.shape
    B, C = continuous.shape
    H0 = W0.shape[1]
    H1 = W1.shape[1]
    N = F * B

    cat_idx = categorical.reshape(F, B).astype(jnp.int32)
    idx = (cat_idx.T + (jnp.arange(F, dtype=jnp.int32) * V)[None, :]).reshape(N)

    gather = _make_gather_kernel(N, D)
    rows, vals = gather(emb2.reshape(F * V, D), emb1.reshape(F * V), idx)
    cat_emb = rows.reshape(B, F * D)
    g1 = vals.reshape(B, F)

    S = jnp.tile(jnp.eye(D, dtype=jnp.float32), (F, 1))
    sc = jnp.concatenate([b_cont, b_out, W_out[0]]).astype(jnp.float32)
    W0c = W0[:C]
    W0e = W0[C:]
    Wh = W_out[1:]

    BB = 2048
    rep = lambda i: (0, 0)
    out = pl.pallas_call(
        _dense_body,
        grid=(B // BB,),
        in_specs=[
            pl.BlockSpec((BB, C), lambda i: (i, 0)),
            pl.BlockSpec((BB, F * D), lambda i: (i, 0)),
            pl.BlockSpec((BB, F), lambda i: (i, 0)),
            pl.BlockSpec((C, 1), rep),
            pl.BlockSpec((C, H0), rep),
            pl.BlockSpec((F * D, H0), rep),
            pl.BlockSpec((1, H0), rep),
            pl.BlockSpec((H0, H1), rep),
            pl.BlockSpec((1, H1), rep),
            pl.BlockSpec((H1, 1), rep),
            pl.BlockSpec((F * D, D), rep),
            pl.BlockSpec(memory_space=pltpu.SMEM),
        ],
        out_specs=pl.BlockSpec((BB, 1), lambda i: (i, 0)),
        out_shape=jax.ShapeDtypeStruct((B, 1), jnp.float32),
    )(continuous, cat_emb, g1, W_cont, W0c, W0e, b0.reshape(1, H0), W1,
      b1.reshape(1, H1), Wh, S, sc)
    return out


# speedup vs baseline: 3.2460x; 3.2460x over previous
import functools

import jax
import jax.numpy as jnp
from jax import lax
from jax.experimental import pallas as pl
from jax.experimental.pallas import tpu as pltpu
from jax.experimental.pallas import tpu_sc as plsc

NW = 32
GCH = 4096


def _make_plane_gather(F, V, D, B):
    FD = F * D
    ppt = FD // NW
    nch = B // GCH
    mesh = plsc.VectorSubcoreMesh(core_axis_name="c", subcore_axis_name="s")

    @functools.partial(
        pl.kernel,
        mesh=mesh,
        compiler_params=pltpu.CompilerParams(needs_layout_passes=False),
        out_type=(
            jax.ShapeDtypeStruct((FD, B), jnp.float32),
            jax.ShapeDtypeStruct((F, B), jnp.float32),
        ),
        scratch_types=[
            pltpu.VMEM((V,), jnp.float32),
            pltpu.VMEM((B,), jnp.int32),
            pltpu.VMEM((GCH,), jnp.float32),
        ],
    )
    def gather(emb2_hbm, emb1_hbm, idx_hbm, out2_hbm, out1_hbm,
               plane_v, idx_v, obuf_v):
        w = lax.axis_index("s") * 2 + lax.axis_index("c")

        def lookup_to(out_hbm, row, fidx):
            pltpu.sync_copy(idx_hbm.at[fidx], idx_v)
            for c in range(nch):
                def gb(i, carry):
                    off = pl.multiple_of(c * GCH + i * 16, 16)
                    iv = idx_v[pl.ds(off, 16)]
                    oo = pl.multiple_of(i * 16, 16)
                    obuf_v[pl.ds(oo, 16)] = plsc.load_gather(plane_v, [iv])
                    return carry
                lax.fori_loop(0, GCH // 16, gb, 0)
                pltpu.sync_copy(obuf_v, out_hbm.at[row, pl.ds(c * GCH, GCH)])

        def body(j, carry):
            p = w * ppt + j
            f = p // D
            pltpu.sync_copy(emb2_hbm.at[p], plane_v)
            lookup_to(out2_hbm, p, f)
            return carry

        lax.fori_loop(0, ppt, body, 0)

        @pl.when(w >= NW - F)
        def _():
            f1 = w - (NW - F)
            pltpu.sync_copy(emb1_hbm.at[f1], plane_v)
            lookup_to(out1_hbm, f1, f1)

    return gather


def _dense_body(cont_ref, catT_ref, g1T_ref, Wc_ref, W0c_ref, W0e_ref,
                b0_ref, W1_ref, b1_ref, Wh_ref, S16_ref, ones1_ref, sc_ref,
                out_ref):
    prec = lax.Precision.HIGHEST
    dn = (((0,), (0,)), ((), ()))
    cont = cont_ref[...]
    catT = catT_ref[...]
    g1T = g1T_ref[...]
    b_cont = sc_ref[0]
    b_out = sc_ref[1]
    w_fm = sc_ref[2]
    fm1 = (jnp.dot(cont, Wc_ref[...], precision=prec)
           + lax.dot_general(g1T, ones1_ref[...][:g1T.shape[0], :], dn,
                             precision=prec)
           + b_cont)
    sum_emb = lax.dot_general(catT, S16_ref[...], dn, precision=prec)
    sumsq = lax.dot_general(catT * catT, ones1_ref[...], dn, precision=prec)
    fm2 = 0.5 * (jnp.sum(sum_emb * sum_emb, axis=1, keepdims=True) - sumsq)
    fm = fm1 + fm2
    h = jnp.maximum(jnp.dot(cont, W0c_ref[...], precision=prec)
                    + lax.dot_general(catT, W0e_ref[...], dn, precision=prec)
                    + b0_ref[...], 0.0)
    h = jnp.maximum(jnp.dot(h, W1_ref[...], precision=prec) + b1_ref[...], 0.0)
    out_ref[...] = (jnp.dot(h, Wh_ref[...], precision=prec)
                    + fm * w_fm + b_out)


def kernel(continuous, categorical, emb1, emb2, W_cont, b_cont, W0, b0, W1,
           b1, W_out, b_out):
    F, V, D = emb2.shape
    B, C = continuous.shape
    H0 = W0.shape[1]
    H1 = W1.shape[1]
    FD = F * D

    emb2_pl = emb2.transpose(0, 2, 1).reshape(FD, V)
    emb1_pl = emb1.transpose(0, 2, 1).reshape(F, V)
    idx = categorical.reshape(F, B).astype(jnp.int32)

    gather = _make_plane_gather(F, V, D, B)
    catT, g1T = gather(emb2_pl, emb1_pl, idx)

    S16 = jnp.tile(jnp.eye(D, dtype=jnp.float32), (F, 1))
    ones1 = jnp.ones((FD, 1), jnp.float32)
    sc = jnp.concatenate([b_cont, b_out, W_out[0]]).astype(jnp.float32)
    W0c = W0[:C]
    W0e = W0[C:]
    Wh = W_out[1:]

    BB = 2048
    rep = lambda i: (0, 0)
    out = pl.pallas_call(
        _dense_body,
        grid=(B // BB,),
        in_specs=[
            pl.BlockSpec((BB, C), lambda i: (i, 0)),
            pl.BlockSpec((FD, BB), lambda i: (0, i)),
            pl.BlockSpec((F, BB), lambda i: (0, i)),
            pl.BlockSpec((C, 1), rep),
            pl.BlockSpec((C, H0), rep),
            pl.BlockSpec((FD, H0), rep),
            pl.BlockSpec((1, H0), rep),
            pl.BlockSpec((H0, H1), rep),
            pl.BlockSpec((1, H1), rep),
            pl.BlockSpec((H1, 1), rep),
            pl.BlockSpec((FD, D), rep),
            pl.BlockSpec((FD, 1), rep),
            pl.BlockSpec(memory_space=pltpu.SMEM),
        ],
        out_specs=pl.BlockSpec((BB, 1), lambda i: (i, 0)),
        out_shape=jax.ShapeDtypeStruct((B, 1), jnp.float32),
    )(continuous, catT, g1T, W_cont, W0c, W0e, b0.reshape(1, H0), W1,
      b1.reshape(1, H1), Wh, S16, ones1, sc)
    return out


# speedup vs baseline: 4.7658x; 1.4682x over previous
import functools

import jax
import jax.numpy as jnp
from jax import lax
from jax.experimental import pallas as pl
from jax.experimental.pallas import tpu as pltpu
from jax.experimental.pallas import tpu_sc as plsc

NW = 32
GCH = 4096


def _make_plane_gather(F, V, D, B):
    FD = F * D
    ppt = FD // NW
    nch = B // GCH
    mesh = plsc.VectorSubcoreMesh(core_axis_name="c", subcore_axis_name="s")

    @functools.partial(
        pl.kernel,
        mesh=mesh,
        compiler_params=pltpu.CompilerParams(needs_layout_passes=False),
        out_type=(
            jax.ShapeDtypeStruct((FD, B), jnp.float32),
            jax.ShapeDtypeStruct((F, B), jnp.float32),
        ),
        scratch_types=[
            pltpu.VMEM((V,), jnp.float32),
            pltpu.VMEM((B,), jnp.int32),
            pltpu.VMEM((GCH,), jnp.float32),
        ],
    )
    def gather(emb2_hbm, emb1_hbm, idx_hbm, out2_hbm, out1_hbm,
               plane_v, idx_v, obuf_v):
        w = lax.axis_index("s") * 2 + lax.axis_index("c")

        def lookup_to(out_hbm, row, fidx):
            pltpu.sync_copy(idx_hbm.at[fidx], idx_v)
            for c in range(nch):
                def gb(i, carry):
                    off = pl.multiple_of(c * GCH + i * 16, 16)
                    iv = idx_v[pl.ds(off, 16)]
                    oo = pl.multiple_of(i * 16, 16)
                    obuf_v[pl.ds(oo, 16)] = plsc.load_gather(plane_v, [iv])
                    return carry
                lax.fori_loop(0, GCH // 16, gb, 0)
                pltpu.sync_copy(obuf_v, out_hbm.at[row, pl.ds(c * GCH, GCH)])

        def body(j, carry):
            p = w * ppt + j
            f = p // D
            pltpu.sync_copy(emb2_hbm.at[p], plane_v)
            lookup_to(out2_hbm, p, f)
            return carry

        lax.fori_loop(0, ppt, body, 0)

        @pl.when(w >= NW - F)
        def _():
            f1 = w - (NW - F)
            pltpu.sync_copy(emb1_hbm.at[f1], plane_v)
            lookup_to(out1_hbm, f1, f1)

    return gather


def _dense_body(cont_ref, catT_ref, g1T_ref, Wc_ref, W0c_ref, W0e_ref,
                b0_ref, W1_ref, b1_ref, Wh_ref, S16_ref, ones1_ref, sc_ref,
                out_ref):
    prec = lax.Precision.DEFAULT
    dn = (((0,), (0,)), ((), ()))
    cont = cont_ref[...]
    catT = catT_ref[...]
    g1T = g1T_ref[...]
    b_cont = sc_ref[0]
    b_out = sc_ref[1]
    w_fm = sc_ref[2]
    fm1 = (jnp.dot(cont, Wc_ref[...], precision=prec)
           + lax.dot_general(g1T, ones1_ref[...][:g1T.shape[0], :], dn,
                             precision=prec)
           + b_cont)
    sum_emb = lax.dot_general(catT, S16_ref[...], dn, precision=prec)
    sumsq = lax.dot_general(catT * catT, ones1_ref[...], dn, precision=prec)
    fm2 = 0.5 * (jnp.sum(sum_emb * sum_emb, axis=1, keepdims=True) - sumsq)
    fm = fm1 + fm2
    h = jnp.maximum(jnp.dot(cont, W0c_ref[...], precision=prec)
                    + lax.dot_general(catT, W0e_ref[...], dn, precision=prec)
                    + b0_ref[...], 0.0)
    h = jnp.maximum(jnp.dot(h, W1_ref[...], precision=prec) + b1_ref[...], 0.0)
    out_ref[...] = (jnp.dot(h, Wh_ref[...], precision=prec)
                    + fm * w_fm + b_out)


def kernel(continuous, categorical, emb1, emb2, W_cont, b_cont, W0, b0, W1,
           b1, W_out, b_out):
    F, V, D = emb2.shape
    B, C = continuous.shape
    H0 = W0.shape[1]
    H1 = W1.shape[1]
    FD = F * D

    emb2_pl = emb2.transpose(0, 2, 1).reshape(FD, V)
    emb1_pl = emb1.transpose(0, 2, 1).reshape(F, V)
    idx = categorical.reshape(F, B).astype(jnp.int32)

    gather = _make_plane_gather(F, V, D, B)
    catT, g1T = gather(emb2_pl, emb1_pl, idx)

    S16 = jnp.tile(jnp.eye(D, dtype=jnp.float32), (F, 1))
    ones1 = jnp.ones((FD, 1), jnp.float32)
    sc = jnp.concatenate([b_cont, b_out, W_out[0]]).astype(jnp.float32)
    W0c = W0[:C]
    W0e = W0[C:]
    Wh = W_out[1:]

    BB = 2048
    rep = lambda i: (0, 0)
    out = pl.pallas_call(
        _dense_body,
        grid=(B // BB,),
        in_specs=[
            pl.BlockSpec((BB, C), lambda i: (i, 0)),
            pl.BlockSpec((FD, BB), lambda i: (0, i)),
            pl.BlockSpec((F, BB), lambda i: (0, i)),
            pl.BlockSpec((C, 1), rep),
            pl.BlockSpec((C, H0), rep),
            pl.BlockSpec((FD, H0), rep),
            pl.BlockSpec((1, H0), rep),
            pl.BlockSpec((H0, H1), rep),
            pl.BlockSpec((1, H1), rep),
            pl.BlockSpec((H1, 1), rep),
            pl.BlockSpec((FD, D), rep),
            pl.BlockSpec((FD, 1), rep),
            pl.BlockSpec(memory_space=pltpu.SMEM),
        ],
        out_specs=pl.BlockSpec((BB, 1), lambda i: (i, 0)),
        out_shape=jax.ShapeDtypeStruct((B, 1), jnp.float32),
    )(continuous, catT, g1T, W_cont, W0c, W0e, b0.reshape(1, H0), W1,
      b1.reshape(1, H1), Wh, S16, ones1, sc)
    return out


# speedup vs baseline: 4.9456x; 1.0377x over previous
import functools

import jax
import jax.numpy as jnp
from jax import lax
from jax.experimental import pallas as pl
from jax.experimental.pallas import tpu as pltpu
from jax.experimental.pallas import tpu_sc as plsc

NW = 32
QB = 4096
U = 4


def _make_plane_gather(F, V, D, B):
    FD = F * D
    ppt = FD // NW
    nq = B // QB
    mesh = plsc.VectorSubcoreMesh(core_axis_name="c", subcore_axis_name="s")

    @functools.partial(
        pl.kernel,
        mesh=mesh,
        compiler_params=pltpu.CompilerParams(needs_layout_passes=False),
        out_type=(
            jax.ShapeDtypeStruct((FD, B), jnp.float32),
            jax.ShapeDtypeStruct((F, B), jnp.float32),
        ),
        scratch_types=[
            pltpu.VMEM((V,), jnp.float32),
            pltpu.VMEM((B,), jnp.int32),
            pltpu.VMEM((2, QB), jnp.float32),
            pltpu.SemaphoreType.DMA,
            pltpu.SemaphoreType.DMA,
        ],
    )
    def gather(emb2_hbm, emb1_hbm, idx_hbm, out2_hbm, out1_hbm,
               plane, idx_v, obuf, semP, semO):
        cid = lax.axis_index("c")
        sid = lax.axis_index("s")
        w = sid * 2 + cid
        is_e1 = w >= NW - F
        f_e1 = w - (NW - F)

        def start_P(tab, row):
            pltpu.async_copy(tab.at[row], plane, semP)

        def wait_P():
            pltpu.make_async_copy(emb2_hbm.at[0], plane, semP).wait()

        def wait_w():
            pltpu.make_async_copy(obuf.at[0],
                                  out2_hbm.at[0].at[pl.ds(0, QB)],
                                  semO).wait()

        def do_quarter(q, out_tab, orow):
            par = q % 2

            def gbody(i, carry):
                o = pl.multiple_of(i * (16 * U), 16)
                for u in range(U):
                    oo = o + u * 16
                    iv = idx_v[pl.ds(q * QB + oo, 16)]
                    obuf[par, pl.ds(oo, 16)] = plsc.load_gather(plane, [iv])
                return carry

            lax.fori_loop(0, QB // (16 * U), gbody, 0)
            pltpu.async_copy(obuf.at[par],
                             out_tab.at[orow].at[pl.ds(q * QB, QB)], semO)

        def plane_proc(out_tab, orow, warm):
            for q in range(nq):
                if q < 2:
                    @pl.when(warm)
                    def _():
                        wait_w()
                else:
                    wait_w()
                do_quarter(q, out_tab, orow)

        def body(j, fprev):
            p = w * ppt + j
            f = p // D
            nxt = j + 1
            pnxt = w * ppt + nxt

            @pl.when(f != fprev)
            def _():
                pltpu.sync_copy(idx_hbm.at[f], idx_v)

            wait_P()
            plane_proc(out2_hbm, p, j > 0)

            @pl.when(nxt < ppt)
            def _():
                start_P(emb2_hbm, pnxt)
            @pl.when(jnp.logical_and(nxt == ppt, is_e1))
            def _():
                start_P(emb1_hbm, f_e1)
            return f

        start_P(emb2_hbm, w * ppt)
        lax.fori_loop(0, ppt, body, -1)

        @pl.when(is_e1)
        def _():
            pltpu.sync_copy(idx_hbm.at[f_e1], idx_v)
            wait_P()
            plane_proc(out1_hbm, f_e1, w >= 0)

        wait_w()
        wait_w()

    return gather


def _dense_body(cont_ref, catT_ref, g1T_ref, Wc_ref, W0c_ref, W0e_ref,
                b0_ref, W1_ref, b1_ref, Wh_ref, S16_ref, ones1_ref, sc_ref,
                out_ref):
    prec = lax.Precision.DEFAULT
    dn = (((0,), (0,)), ((), ()))
    cont = cont_ref[...]
    catT = catT_ref[...]
    g1T = g1T_ref[...]
    b_cont = sc_ref[0]
    b_out = sc_ref[1]
    w_fm = sc_ref[2]
    fm1 = (jnp.dot(cont, Wc_ref[...], precision=prec)
           + lax.dot_general(g1T, ones1_ref[...][:g1T.shape[0], :], dn,
                             precision=prec)
           + b_cont)
    sum_emb = lax.dot_general(catT, S16_ref[...], dn, precision=prec)
    sumsq = lax.dot_general(catT * catT, ones1_ref[...], dn, precision=prec)
    fm2 = 0.5 * (jnp.sum(sum_emb * sum_emb, axis=1, keepdims=True) - sumsq)
    fm = fm1 + fm2
    h = jnp.maximum(jnp.dot(cont, W0c_ref[...], precision=prec)
                    + lax.dot_general(catT, W0e_ref[...], dn, precision=prec)
                    + b0_ref[...], 0.0)
    h = jnp.maximum(jnp.dot(h, W1_ref[...], precision=prec) + b1_ref[...], 0.0)
    out_ref[...] = (jnp.dot(h, Wh_ref[...], precision=prec)
                    + fm * w_fm + b_out)


def kernel(continuous, categorical, emb1, emb2, W_cont, b_cont, W0, b0, W1,
           b1, W_out, b_out):
    F, V, D = emb2.shape
    B, C = continuous.shape
    H0 = W0.shape[1]
    H1 = W1.shape[1]
    FD = F * D

    emb2_pl = emb2.transpose(0, 2, 1).reshape(FD, V)
    emb1_pl = emb1.transpose(0, 2, 1).reshape(F, V)
    idx = categorical.reshape(F, B).astype(jnp.int32)

    gather = _make_plane_gather(F, V, D, B)
    catT, g1T = gather(emb2_pl, emb1_pl, idx)

    S16 = jnp.tile(jnp.eye(D, dtype=jnp.float32), (F, 1))
    ones1 = jnp.ones((FD, 1), jnp.float32)
    sc = jnp.concatenate([b_cont, b_out, W_out[0]]).astype(jnp.float32)
    W0c = W0[:C]
    W0e = W0[C:]
    Wh = W_out[1:]

    BB = 2048
    rep = lambda i: (0, 0)
    out = pl.pallas_call(
        _dense_body,
        grid=(B // BB,),
        in_specs=[
            pl.BlockSpec((BB, C), lambda i: (i, 0)),
            pl.BlockSpec((FD, BB), lambda i: (0, i)),
            pl.BlockSpec((F, BB), lambda i: (0, i)),
            pl.BlockSpec((C, 1), rep),
            pl.BlockSpec((C, H0), rep),
            pl.BlockSpec((FD, H0), rep),
            pl.BlockSpec((1, H0), rep),
            pl.BlockSpec((H0, H1), rep),
            pl.BlockSpec((1, H1), rep),
            pl.BlockSpec((H1, 1), rep),
            pl.BlockSpec((FD, D), rep),
            pl.BlockSpec((FD, 1), rep),
            pl.BlockSpec(memory_space=pltpu.SMEM),
        ],
        out_specs=pl.BlockSpec((BB, 1), lambda i: (i, 0)),
        out_shape=jax.ShapeDtypeStruct((B, 1), jnp.float32),
    )(continuous, catT, g1T, W_cont, W0c, W0e, b0.reshape(1, H0), W1,
      b1.reshape(1, H1), Wh, S16, ones1, sc)
    return out


# speedup vs baseline: 7.8731x; 1.5919x over previous
import functools

import jax
import jax.numpy as jnp
from jax import lax
from jax.experimental import pallas as pl
from jax.experimental.pallas import tpu as pltpu
from jax.experimental.pallas import tpu_sc as plsc

NW = 32
QB = 4096
U = 4


def _make_plane_gather(F, V, D, B):
    FD = F * D
    ppt = FD // NW
    nq = B // QB
    mesh = plsc.VectorSubcoreMesh(core_axis_name="c", subcore_axis_name="s")

    @functools.partial(
        pl.kernel,
        mesh=mesh,
        compiler_params=pltpu.CompilerParams(needs_layout_passes=False),
        out_type=(
            jax.ShapeDtypeStruct((FD, B), jnp.float32),
            jax.ShapeDtypeStruct((F, B), jnp.float32),
        ),
        scratch_types=[
            pltpu.VMEM((V,), jnp.float32),
            pltpu.VMEM((B,), jnp.int32),
            pltpu.VMEM((2, QB), jnp.float32),
            pltpu.SemaphoreType.DMA,
            pltpu.SemaphoreType.DMA,
        ],
    )
    def gather(emb2_hbm, emb1_hbm, idx_hbm, out2_hbm, out1_hbm,
               plane, idx_v, obuf, semP, semO):
        cid = lax.axis_index("c")
        sid = lax.axis_index("s")
        w = sid * 2 + cid
        is_e1 = w >= NW - F
        f_e1 = w - (NW - F)

        def start_P(tab, row):
            pltpu.async_copy(tab.at[row], plane, semP)

        def wait_P():
            pltpu.make_async_copy(emb2_hbm.at[0], plane, semP).wait()

        def wait_w():
            pltpu.make_async_copy(obuf.at[0],
                                  out2_hbm.at[0].at[pl.ds(0, QB)],
                                  semO).wait()

        def do_quarter(q, out_tab, orow):
            par = q % 2

            @plsc.parallel_loop(0, QB // 16, unroll=U)
            def _(i):
                o = pl.multiple_of(i * 16, 16)
                iv = idx_v[pl.ds(q * QB + o, 16)]
                obuf[par, pl.ds(o, 16)] = plsc.load_gather(plane, [iv])
            pltpu.async_copy(obuf.at[par],
                             out_tab.at[orow].at[pl.ds(q * QB, QB)], semO)

        def plane_proc(out_tab, orow, warm):
            for q in range(nq):
                if q < 2:
                    @pl.when(warm)
                    def _():
                        wait_w()
                else:
                    wait_w()
                do_quarter(q, out_tab, orow)

        def body(j, fprev):
            p = w * ppt + j
            f = p // D
            nxt = j + 1
            pnxt = w * ppt + nxt

            @pl.when(f != fprev)
            def _():
                pltpu.sync_copy(idx_hbm.at[f], idx_v)

            wait_P()
            plane_proc(out2_hbm, p, j > 0)

            @pl.when(nxt < ppt)
            def _():
                start_P(emb2_hbm, pnxt)
            @pl.when(jnp.logical_and(nxt == ppt, is_e1))
            def _():
                start_P(emb1_hbm, f_e1)
            return f

        start_P(emb2_hbm, w * ppt)
        lax.fori_loop(0, ppt, body, -1)

        @pl.when(is_e1)
        def _():
            pltpu.sync_copy(idx_hbm.at[f_e1], idx_v)
            wait_P()
            plane_proc(out1_hbm, f_e1, w >= 0)

        wait_w()
        wait_w()

    return gather


def _dense_body(cont_ref, catT_ref, g1T_ref, Wc_ref, W0c_ref, W0e_ref,
                b0_ref, W1_ref, b1_ref, Wh_ref, S16_ref, ones1_ref, sc_ref,
                out_ref):
    prec = lax.Precision.DEFAULT
    dn = (((0,), (0,)), ((), ()))
    cont = cont_ref[...]
    catT = catT_ref[...]
    g1T = g1T_ref[...]
    b_cont = sc_ref[0]
    b_out = sc_ref[1]
    w_fm = sc_ref[2]
    fm1 = (jnp.dot(cont, Wc_ref[...], precision=prec)
           + lax.dot_general(g1T, ones1_ref[...][:g1T.shape[0], :], dn,
                             precision=prec)
           + b_cont)
    sum_emb = lax.dot_general(catT, S16_ref[...], dn, precision=prec)
    sumsq = lax.dot_general(catT * catT, ones1_ref[...], dn, precision=prec)
    fm2 = 0.5 * (jnp.sum(sum_emb * sum_emb, axis=1, keepdims=True) - sumsq)
    fm = fm1 + fm2
    h = jnp.maximum(jnp.dot(cont, W0c_ref[...], precision=prec)
                    + lax.dot_general(catT, W0e_ref[...], dn, precision=prec)
                    + b0_ref[...], 0.0)
    h = jnp.maximum(jnp.dot(h, W1_ref[...], precision=prec) + b1_ref[...], 0.0)
    out_ref[...] = (jnp.dot(h, Wh_ref[...], precision=prec)
                    + fm * w_fm + b_out)


def kernel(continuous, categorical, emb1, emb2, W_cont, b_cont, W0, b0, W1,
           b1, W_out, b_out):
    F, V, D = emb2.shape
    B, C = continuous.shape
    H0 = W0.shape[1]
    H1 = W1.shape[1]
    FD = F * D

    emb2_pl = emb2.transpose(0, 2, 1).reshape(FD, V)
    emb1_pl = emb1.transpose(0, 2, 1).reshape(F, V)
    idx = categorical.reshape(F, B).astype(jnp.int32)

    gather = _make_plane_gather(F, V, D, B)
    catT, g1T = gather(emb2_pl, emb1_pl, idx)

    S16 = jnp.tile(jnp.eye(D, dtype=jnp.float32), (F, 1))
    ones1 = jnp.ones((FD, 1), jnp.float32)
    sc = jnp.concatenate([b_cont, b_out, W_out[0]]).astype(jnp.float32)
    W0c = W0[:C]
    W0e = W0[C:]
    Wh = W_out[1:]

    BB = 2048
    rep = lambda i: (0, 0)
    out = pl.pallas_call(
        _dense_body,
        grid=(B // BB,),
        in_specs=[
            pl.BlockSpec((BB, C), lambda i: (i, 0)),
            pl.BlockSpec((FD, BB), lambda i: (0, i)),
            pl.BlockSpec((F, BB), lambda i: (0, i)),
            pl.BlockSpec((C, 1), rep),
            pl.BlockSpec((C, H0), rep),
            pl.BlockSpec((FD, H0), rep),
            pl.BlockSpec((1, H0), rep),
            pl.BlockSpec((H0, H1), rep),
            pl.BlockSpec((1, H1), rep),
            pl.BlockSpec((H1, 1), rep),
            pl.BlockSpec((FD, D), rep),
            pl.BlockSpec((FD, 1), rep),
            pl.BlockSpec(memory_space=pltpu.SMEM),
        ],
        out_specs=pl.BlockSpec((BB, 1), lambda i: (i, 0)),
        out_shape=jax.ShapeDtypeStruct((B, 1), jnp.float32),
    )(continuous, catT, g1T, W_cont, W0c, W0e, b0.reshape(1, H0), W1,
      b1.reshape(1, H1), Wh, S16, ones1, sc)
    return out
